# Initial kernel scaffold; baseline (speedup 1.0000x reference)
import jax
import jax.numpy as jnp
from jax.experimental import pallas as pl


def kernel(x, edge_index, batch, node_attr, edge_attr, glob_attr, se_W, se_b, ne_W, ne_b, ee_W, ee_b, ge_W, ge_b, pe_W1, pe_b1, pe_W2, pe_b2, pe_W3, pe_b3, pv_W1, pv_b1, pv_W2, pv_b2, pv_W3, pv_b3, dec_W, dec_b):
    raise NotImplementedError("write your pallas kernel here")


# speedup vs baseline: 2.8280x; 2.8280x over previous
import functools

import jax
import jax.numpy as jnp
from jax import lax
from jax.experimental import pallas as pl
from jax.experimental.pallas import tpu as pltpu
from jax.experimental.pallas import tpu_sc as plsc

N = 50000
E = 800000
TW = 32
NP = 50048
NROWS = NP // 16

NW = 32
GPW = 50176
EPAD = 16 * GPW
GK = 8
GC = GPW // (GK * 128)

SPW = 25600
ESC = NW * SPW
SK = 8
SCH = SPW // (SK * 128)
DUMMY = N + 8


def _node_encoder_body(x_ref, na_ref, seW_ref, seb_ref, neW_ref, neb_ref,
                       out_ref):
    xb = x_ref[...]
    sin_e = jnp.maximum(jnp.sin(xb) * seW_ref[0:1, :] + seb_ref[0:1, :], 0.0)
    cos_e = jnp.maximum(jnp.cos(xb) * seW_ref[1:2, :] + seb_ref[1:2, :], 0.0)
    ne = jnp.maximum(na_ref[...] * neW_ref[...] + neb_ref[...], 0.0)
    out_ref[...] = jnp.concatenate(
        [sin_e, cos_e, ne, jnp.zeros_like(ne)], axis=-1)


def _node_encoder(x, node_attr, se_W, se_b, ne_W, ne_b):
    bn = 10000
    grid = N // bn
    return pl.pallas_call(
        _node_encoder_body,
        grid=(grid,),
        in_specs=[
            pl.BlockSpec((bn, 1), lambda i: (i, 0)),
            pl.BlockSpec((bn, 1), lambda i: (i, 0)),
            pl.BlockSpec(se_W.shape, lambda i: (0, 0)),
            pl.BlockSpec(se_b.shape, lambda i: (0, 0)),
            pl.BlockSpec(ne_W.shape, lambda i: (0, 0)),
            pl.BlockSpec(ne_b.shape, lambda i: (0, 0)),
        ],
        out_specs=pl.BlockSpec((bn, TW), lambda i: (i, 0)),
        out_shape=jax.ShapeDtypeStruct((N, TW), jnp.float32),
    )(x, node_attr, se_W, se_b, ne_W, ne_b)


@functools.cache
def _sc_kernels():
    mesh = plsc.VectorSubcoreMesh(core_axis_name="c", subcore_axis_name="s")

    @functools.partial(
        pl.kernel,
        out_type=jax.ShapeDtypeStruct((NW, GPW, TW), jnp.float32),
        mesh=mesh,
        scratch_types=[
            pltpu.VMEM((GK, 128), jnp.int32),
            pltpu.VMEM((GK * 128, TW), jnp.float32),
            pltpu.SemaphoreType.DMA,
        ],
        compiler_params=pltpu.CompilerParams(use_tc_tiling_on_sc=False),
    )
    def _gather_k(table_hbm, idx_hbm, out_hbm, idx_v, rows_v, sem):
        w = lax.axis_index("s") * 2 + lax.axis_index("c")

        def chunk(c, carry):
            pltpu.sync_copy(idx_hbm.at[w, pl.ds(c * GK, GK)], idx_v)
            cps = [
                pltpu.async_copy(table_hbm.at[idx_v.at[j]],
                                 rows_v.at[pl.ds(j * 128, 128)], sem)
                for j in range(GK)
            ]
            for cp in cps:
                cp.wait()
            pltpu.sync_copy(rows_v,
                            out_hbm.at[w, pl.ds(c * GK * 128, GK * 128)])
            return carry

        lax.fori_loop(0, GC, chunk, 0)

    @functools.partial(
        pl.kernel,
        out_type=jax.ShapeDtypeStruct((2, NP, 8), jnp.float32),
        mesh=mesh,
        scratch_types=[
            pltpu.VMEM((SK, 128), jnp.int32),
            pltpu.VMEM((SK * 128, 8), jnp.float32),
            pltpu.VMEM_SHARED((NP, 8), jnp.float32),
        ],
        compiler_params=pltpu.CompilerParams(use_tc_tiling_on_sc=False),
    )
    def _scatter_k(ep_hbm, cidx_hbm, zero_hbm, out_hbm, idx_v, ep_v, acc):
        cid = lax.axis_index("c")
        sid = lax.axis_index("s")
        w = sid * 2 + cid
        pltpu.sync_copy(zero_hbm.at[pl.ds(sid * NROWS, NROWS)],
                        acc.at[pl.ds(sid * NROWS, NROWS)])
        plsc.subcore_barrier()

        def chunk(c, carry):
            base = w * SPW + c * (SK * 128)
            pltpu.sync_copy(cidx_hbm.at[w, pl.ds(c * SK, SK)], idx_v)
            pltpu.sync_copy(ep_hbm.at[pl.ds(base, SK * 128)], ep_v)
            for j in range(SK):
                pltpu.sync_copy(ep_v.at[pl.ds(j * 128, 128)],
                                acc.at[idx_v.at[j]], add=True)
            return carry

        lax.fori_loop(0, SCH, chunk, 0)
        plsc.subcore_barrier()
        pltpu.sync_copy(acc.at[pl.ds(sid * NROWS, NROWS)],
                        out_hbm.at[cid, pl.ds(sid * NROWS, NROWS)])

    return _gather_k, _scatter_k


def _edge_mlp_body(gr_ref, gc_ref, ea_ref, eeW_ref, eeb_ref, W1r_ref, W1c_ref,
                   W1e_ref, b1_ref, W2_ref, b2_ref, W3_ref, b3_ref, out_ref):
    gr = gr_ref[0]
    gc = gc_ref[0]
    eemb = jnp.maximum(ea_ref[...] * eeW_ref[...] + eeb_ref[...], 0.0)
    h = (jnp.dot(gr, W1r_ref[...], preferred_element_type=jnp.float32)
         + jnp.dot(gc, W1c_ref[...], preferred_element_type=jnp.float32)
         + jnp.dot(eemb, W1e_ref[...], preferred_element_type=jnp.float32)
         + b1_ref[...])
    h = jnp.maximum(h, 0.0)
    h = jnp.maximum(
        jnp.dot(h, W2_ref[...], preferred_element_type=jnp.float32)
        + b2_ref[...], 0.0)
    out_ref[...] = (jnp.dot(h, W3_ref[...], preferred_element_type=jnp.float32)
                    + b3_ref[...])


def _edge_mlp(gath, edge_attr, eeW, eeb, W1r, W1c, W1e, b1, W2, b2, W3, b3):
    be = 8000
    grid = E // be
    full = lambda a: pl.BlockSpec(a.shape, lambda i: tuple(0 for _ in a.shape))
    return pl.pallas_call(
        _edge_mlp_body,
        grid=(grid,),
        in_specs=[
            pl.BlockSpec((1, be, TW), lambda i: (0, i, 0)),
            pl.BlockSpec((1, be, TW), lambda i: (1, i, 0)),
            pl.BlockSpec((be, 1), lambda i: (i, 0)),
            full(eeW), full(eeb), full(W1r), full(W1c), full(W1e), full(b1),
            full(W2), full(b2), full(W3), full(b3),
        ],
        out_specs=pl.BlockSpec((be, 8), lambda i: (i, 0)),
        out_shape=jax.ShapeDtypeStruct((ESC, 8), jnp.float32),
    )(gath, gath, edge_attr, eeW, eeb, W1r, W1c, W1e, b1, W2, b2, W3, b3)


def _node_mlp_body(p_ref, W1_ref, b1_ref, W2_ref, b2_ref, Wd_ref, bd_ref,
                   out_ref):
    eb = p_ref[0] + p_ref[1]
    g = jnp.maximum(
        jnp.dot(eb, W1_ref[...], preferred_element_type=jnp.float32)
        + b1_ref[...], 0.0)
    g = jnp.maximum(
        jnp.dot(g, W2_ref[...], preferred_element_type=jnp.float32)
        + b2_ref[...], 0.0)
    out_ref[...] = (jnp.dot(g, Wd_ref[...], preferred_element_type=jnp.float32)
                    + bd_ref[...])


def _node_mlp(parts, W1, b1, W2, b2, Wd, bd):
    bn = 10000
    grid = N // bn
    full = lambda a: pl.BlockSpec(a.shape, lambda i: tuple(0 for _ in a.shape))
    return pl.pallas_call(
        _node_mlp_body,
        grid=(grid,),
        in_specs=[
            pl.BlockSpec((2, bn, 8), lambda i: (0, i, 0)),
            full(W1), full(b1), full(W2), full(b2), full(Wd), full(bd),
        ],
        out_specs=pl.BlockSpec((bn, 1), lambda i: (i, 0)),
        out_shape=jax.ShapeDtypeStruct((N, 1), jnp.float32),
    )(parts, W1, b1, W2, b2, Wd, bd)


def kernel(x, edge_index, batch, node_attr, edge_attr, glob_attr,
           se_W, se_b, ne_W, ne_b, ee_W, ee_b, ge_W, ge_b,
           pe_W1, pe_b1, pe_W2, pe_b2, pe_W3, pe_b3,
           pv_W1, pv_b1, pv_W2, pv_b2, pv_W3, pv_b3,
           dec_W, dec_b):
    gidx = jnp.pad(edge_index, ((0, 0), (0, EPAD - E)))
    gidx = gidx.reshape(NW, GPW // 128, 128)
    cidx = jnp.pad(edge_index[1], (0, ESC - E), constant_values=DUMMY)
    cidx = cidx.reshape(NW, SPW // 128, 128)
    zeros = jnp.zeros((NP, 8), jnp.float32)

    z8 = jnp.zeros((8, pe_W1.shape[1]), jnp.float32)
    W1r = jnp.concatenate([pe_W1[0:24], z8], axis=0)
    W1c = jnp.concatenate([pe_W1[24:48], z8], axis=0)
    W1e = pe_W1[48:56]
    Wd = pv_W3 @ dec_W
    bd = (pv_b3 @ dec_W + dec_b).reshape(1, 1)

    gather_k, scatter_k = _sc_kernels()
    table = _node_encoder(x, node_attr, se_W, se_b,
                          ne_W, ne_b.reshape(1, -1))
    gath = gather_k(table, gidx).reshape(2, EPAD, TW)
    ep = _edge_mlp(gath, edge_attr, ee_W, ee_b.reshape(1, -1),
                   W1r, W1c, W1e, pe_b1.reshape(1, -1),
                   pe_W2, pe_b2.reshape(1, -1), pe_W3, pe_b3.reshape(1, -1))
    parts = scatter_k(ep, cidx, zeros)
    out = _node_mlp(parts, pv_W1, pv_b1.reshape(1, -1),
                    pv_W2, pv_b2.reshape(1, -1), Wd, bd)
    return out


# speedup vs baseline: 2.8478x; 1.0070x over previous
import functools

import jax
import jax.numpy as jnp
from jax import lax
from jax.experimental import pallas as pl
from jax.experimental.pallas import tpu as pltpu
from jax.experimental.pallas import tpu_sc as plsc

N = 50000
E = 800000
TW = 32
NP = 50048
NROWS = NP // 16

NW = 32
GPW = E // 16
GK = 8
GCH = GK * 128
GC = GPW // GCH
GT = GPW - GC * GCH
GTN = GT // 128
GTR = GT - GTN * 128

SPW = E // NW
SROWS = 196
SK = 8
SCH = SROWS // SK
STL = SROWS - SCH * SK
EEP = E + 128
DUMMY = N + 8


def _node_encoder_body(x_ref, na_ref, seW_ref, seb_ref, neW_ref, neb_ref,
                       out_ref):
    xb = x_ref[...]
    sin_e = jnp.maximum(jnp.sin(xb) * seW_ref[0:1, :] + seb_ref[0:1, :], 0.0)
    cos_e = jnp.maximum(jnp.cos(xb) * seW_ref[1:2, :] + seb_ref[1:2, :], 0.0)
    ne = jnp.maximum(na_ref[...] * neW_ref[...] + neb_ref[...], 0.0)
    out_ref[...] = jnp.concatenate(
        [sin_e, cos_e, ne, jnp.zeros_like(ne)], axis=-1)


def _node_encoder(x, node_attr, se_W, se_b, ne_W, ne_b):
    bn = 10000
    grid = N // bn
    return pl.pallas_call(
        _node_encoder_body,
        grid=(grid,),
        in_specs=[
            pl.BlockSpec((bn, 1), lambda i: (i, 0)),
            pl.BlockSpec((bn, 1), lambda i: (i, 0)),
            pl.BlockSpec(se_W.shape, lambda i: (0, 0)),
            pl.BlockSpec(se_b.shape, lambda i: (0, 0)),
            pl.BlockSpec(ne_W.shape, lambda i: (0, 0)),
            pl.BlockSpec(ne_b.shape, lambda i: (0, 0)),
        ],
        out_specs=pl.BlockSpec((bn, TW), lambda i: (i, 0)),
        out_shape=jax.ShapeDtypeStruct((N, TW), jnp.float32),
    )(x, node_attr, se_W, se_b, ne_W, ne_b)


@functools.cache
def _sc_kernels():
    mesh = plsc.VectorSubcoreMesh(core_axis_name="c", subcore_axis_name="s")

    @functools.partial(
        pl.kernel,
        out_type=jax.ShapeDtypeStruct((2, E, TW), jnp.float32),
        mesh=mesh,
        scratch_types=[
            pltpu.VMEM((GCH,), jnp.int32),
            pltpu.VMEM((GCH, TW), jnp.float32),
            pltpu.SemaphoreType.DMA,
        ],
        compiler_params=pltpu.CompilerParams(use_tc_tiling_on_sc=False),
    )
    def _gather_k(table_hbm, eidx_hbm, out_hbm, idx_v, rows_v, sem):
        w = lax.axis_index("s") * 2 + lax.axis_index("c")
        side = w // 16
        base = (w % 16) * GPW

        def do_block(c, nrows, ndma, last):
            pltpu.sync_copy(eidx_hbm.at[side, pl.ds(base + c * GCH, nrows)],
                            idx_v.at[pl.ds(0, nrows)])
            cps = [
                pltpu.async_copy(
                    table_hbm.at[idx_v.at[pl.ds(j * 128,
                                                last if j == ndma - 1 else 128)]],
                    rows_v.at[pl.ds(j * 128,
                                    last if j == ndma - 1 else 128)], sem)
                for j in range(ndma)
            ]
            for cp in cps:
                cp.wait()
            pltpu.sync_copy(rows_v.at[pl.ds(0, nrows)],
                            out_hbm.at[side, pl.ds(base + c * GCH, nrows)])

        def chunk(c, carry):
            do_block(c, GCH, GK, 128)
            return carry

        lax.fori_loop(0, GC, chunk, 0)
        do_block(GC, GT, GTN + 1, GTR)

    @functools.partial(
        pl.kernel,
        out_type=jax.ShapeDtypeStruct((2, NP, 8), jnp.float32),
        mesh=mesh,
        scratch_types=[
            pltpu.VMEM((SK, 128), jnp.int32),
            pltpu.VMEM((SK * 128, 8), jnp.float32),
            pltpu.VMEM_SHARED((NP, 8), jnp.float32),
        ],
        compiler_params=pltpu.CompilerParams(use_tc_tiling_on_sc=False),
    )
    def _scatter_k(ep_hbm, cidx_hbm, zero_hbm, out_hbm, idx_v, ep_v, acc):
        cid = lax.axis_index("c")
        sid = lax.axis_index("s")
        w = sid * 2 + cid
        pltpu.sync_copy(zero_hbm.at[pl.ds(sid * NROWS, NROWS)],
                        acc.at[pl.ds(sid * NROWS, NROWS)])
        plsc.subcore_barrier()

        def do_block(c, nidx):
            base = w * SPW + c * (SK * 128)
            pltpu.sync_copy(cidx_hbm.at[w, pl.ds(c * SK, nidx)],
                            idx_v.at[pl.ds(0, nidx)])
            pltpu.sync_copy(ep_hbm.at[pl.ds(base, nidx * 128)],
                            ep_v.at[pl.ds(0, nidx * 128)])
            for j in range(nidx):
                pltpu.sync_copy(ep_v.at[pl.ds(j * 128, 128)],
                                acc.at[idx_v.at[j]], add=True)

        def chunk(c, carry):
            do_block(c, SK)
            return carry

        lax.fori_loop(0, SCH, chunk, 0)
        do_block(SCH, STL)
        plsc.subcore_barrier()
        pltpu.sync_copy(acc.at[pl.ds(sid * NROWS, NROWS)],
                        out_hbm.at[cid, pl.ds(sid * NROWS, NROWS)])

    return _gather_k, _scatter_k


def _edge_mlp_body(gr_ref, gc_ref, ea_ref, eeW_ref, eeb_ref, W1r_ref, W1c_ref,
                   W1e_ref, b1_ref, W2_ref, b2_ref, W3_ref, b3_ref, out_ref):
    gr = gr_ref[0]
    gc = gc_ref[0]
    eemb = jnp.maximum(ea_ref[...] * eeW_ref[...] + eeb_ref[...], 0.0)
    h = (jnp.dot(gr, W1r_ref[...], preferred_element_type=jnp.float32)
         + jnp.dot(gc, W1c_ref[...], preferred_element_type=jnp.float32)
         + jnp.dot(eemb, W1e_ref[...], preferred_element_type=jnp.float32)
         + b1_ref[...])
    h = jnp.maximum(h, 0.0)
    h = jnp.maximum(
        jnp.dot(h, W2_ref[...], preferred_element_type=jnp.float32)
        + b2_ref[...], 0.0)
    out_ref[...] = (jnp.dot(h, W3_ref[...], preferred_element_type=jnp.float32)
                    + b3_ref[...])


def _edge_mlp(gath, edge_attr, eeW, eeb, W1r, W1c, W1e, b1, W2, b2, W3, b3):
    be = 8000
    grid = E // be
    full = lambda a: pl.BlockSpec(a.shape, lambda i: tuple(0 for _ in a.shape))
    return pl.pallas_call(
        _edge_mlp_body,
        grid=(grid,),
        in_specs=[
            pl.BlockSpec((1, be, TW), lambda i: (0, i, 0)),
            pl.BlockSpec((1, be, TW), lambda i: (1, i, 0)),
            pl.BlockSpec((be, 1), lambda i: (i, 0)),
            full(eeW), full(eeb), full(W1r), full(W1c), full(W1e), full(b1),
            full(W2), full(b2), full(W3), full(b3),
        ],
        out_specs=pl.BlockSpec((be, 8), lambda i: (i, 0)),
        out_shape=jax.ShapeDtypeStruct((EEP, 8), jnp.float32),
    )(gath, gath, edge_attr, eeW, eeb, W1r, W1c, W1e, b1, W2, b2, W3, b3)


def _node_mlp_body(p_ref, W1_ref, b1_ref, W2_ref, b2_ref, Wd_ref, bd_ref,
                   out_ref):
    eb = p_ref[0] + p_ref[1]
    g = jnp.maximum(
        jnp.dot(eb, W1_ref[...], preferred_element_type=jnp.float32)
        + b1_ref[...], 0.0)
    g = jnp.maximum(
        jnp.dot(g, W2_ref[...], preferred_element_type=jnp.float32)
        + b2_ref[...], 0.0)
    out_ref[...] = (jnp.dot(g, Wd_ref[...], preferred_element_type=jnp.float32)
                    + bd_ref[...])


def _node_mlp(parts, W1, b1, W2, b2, Wd, bd):
    bn = 10000
    grid = N // bn
    full = lambda a: pl.BlockSpec(a.shape, lambda i: tuple(0 for _ in a.shape))
    return pl.pallas_call(
        _node_mlp_body,
        grid=(grid,),
        in_specs=[
            pl.BlockSpec((2, bn, 8), lambda i: (0, i, 0)),
            full(W1), full(b1), full(W2), full(b2), full(Wd), full(bd),
        ],
        out_specs=pl.BlockSpec((bn, 1), lambda i: (i, 0)),
        out_shape=jax.ShapeDtypeStruct((N, 1), jnp.float32),
    )(parts, W1, b1, W2, b2, Wd, bd)


def kernel(x, edge_index, batch, node_attr, edge_attr, glob_attr,
           se_W, se_b, ne_W, ne_b, ee_W, ee_b, ge_W, ge_b,
           pe_W1, pe_b1, pe_W2, pe_b2, pe_W3, pe_b3,
           pv_W1, pv_b1, pv_W2, pv_b2, pv_W3, pv_b3,
           dec_W, dec_b):
    cidx = jnp.pad(edge_index[1].reshape(NW, SPW),
                   ((0, 0), (0, SROWS * 128 - SPW)), constant_values=DUMMY)
    cidx = cidx.reshape(NW, SROWS, 128)
    zeros = jnp.zeros((NP, 8), jnp.float32)

    z8 = jnp.zeros((8, pe_W1.shape[1]), jnp.float32)
    W1r = jnp.concatenate([pe_W1[0:24], z8], axis=0)
    W1c = jnp.concatenate([pe_W1[24:48], z8], axis=0)
    W1e = pe_W1[48:56]
    Wd = pv_W3 @ dec_W
    bd = (pv_b3 @ dec_W + dec_b).reshape(1, 1)

    gather_k, scatter_k = _sc_kernels()
    table = _node_encoder(x, node_attr, se_W, se_b,
                          ne_W, ne_b.reshape(1, -1))
    gath = gather_k(table, edge_index)
    ep = _edge_mlp(gath, edge_attr, ee_W, ee_b.reshape(1, -1),
                   W1r, W1c, W1e, pe_b1.reshape(1, -1),
                   pe_W2, pe_b2.reshape(1, -1), pe_W3, pe_b3.reshape(1, -1))
    parts = scatter_k(ep, cidx, zeros)
    out = _node_mlp(parts, pv_W1, pv_b1.reshape(1, -1),
                    pv_W2, pv_b2.reshape(1, -1), Wd, bd)
    return out


# speedup vs baseline: 4.7136x; 1.6552x over previous
import functools

import jax
import jax.numpy as jnp
from jax import lax
from jax.experimental import pallas as pl
from jax.experimental.pallas import tpu as pltpu
from jax.experimental.pallas import tpu_sc as plsc

N = 50000
E = 800000
TW = 32
NP = 50048
NROWS = NP // 16

NW = 32
GPW = E // 16
GK = 8
GCH = GK * 128
GC = GPW // GCH
GT = GPW - GC * GCH
GTN = GT // 128
GTR = GT - GTN * 128

SPW = E // NW
SROWS = 196
SK = 8
SCH = SROWS // SK
STL = SROWS - SCH * SK
EEP = E + 128
DUMMY = N + 8


def _node_encoder_body(x_ref, na_ref, seW_ref, seb_ref, neW_ref, neb_ref,
                       out_ref):
    xb = x_ref[...]
    nab = na_ref[...]
    s = jnp.sin(xb)
    c = jnp.cos(xb)
    pieces = []
    for k in range(4):
        sin_e = jnp.maximum(
            s[:, k:k + 1] * seW_ref[0:1, :] + seb_ref[0:1, :], 0.0)
        cos_e = jnp.maximum(
            c[:, k:k + 1] * seW_ref[1:2, :] + seb_ref[1:2, :], 0.0)
        ne = jnp.maximum(
            nab[:, k:k + 1] * neW_ref[...] + neb_ref[...], 0.0)
        pieces += [sin_e, cos_e, ne, jnp.zeros_like(ne)]
    out_ref[...] = jnp.concatenate(pieces, axis=-1)


def _node_encoder(x4, na4, se_W, se_b, ne_W, ne_b):
    br = 1000
    grid = -(-(N // 4) // br)
    return pl.pallas_call(
        _node_encoder_body,
        grid=(grid,),
        in_specs=[
            pl.BlockSpec((br, 4), lambda i: (i, 0)),
            pl.BlockSpec((br, 4), lambda i: (i, 0)),
            pl.BlockSpec(se_W.shape, lambda i: (0, 0)),
            pl.BlockSpec(se_b.shape, lambda i: (0, 0)),
            pl.BlockSpec(ne_W.shape, lambda i: (0, 0)),
            pl.BlockSpec(ne_b.shape, lambda i: (0, 0)),
        ],
        out_specs=pl.BlockSpec((br, 128), lambda i: (i, 0)),
        out_shape=jax.ShapeDtypeStruct((N // 4, 128), jnp.float32),
    )(x4, na4, se_W, se_b, ne_W, ne_b)


@functools.cache
def _sc_kernels():
    mesh = plsc.VectorSubcoreMesh(core_axis_name="c", subcore_axis_name="s")

    @functools.partial(
        pl.kernel,
        out_type=jax.ShapeDtypeStruct((2, E, TW), jnp.float32),
        mesh=mesh,
        scratch_types=[
            pltpu.VMEM((2, GCH), jnp.int32),
            pltpu.VMEM((2, GCH, TW), jnp.float32),
            pltpu.SemaphoreType.DMA,
            pltpu.SemaphoreType.DMA,
        ],
        compiler_params=pltpu.CompilerParams(use_tc_tiling_on_sc=False),
    )
    def _gather_k(table_hbm, eidx_hbm, out_hbm, idx_v, rows_v, sem0, sem1):
        w = lax.axis_index("s") * 2 + lax.axis_index("c")
        side = w // 16
        base = (w % 16) * GPW
        sems = (sem0, sem1)

        def fire(c, b, nrows, ndma, last):
            pltpu.sync_copy(eidx_hbm.at[side, pl.ds(base + c * GCH, nrows)],
                            idx_v.at[b, pl.ds(0, nrows)])
            for j in range(ndma):
                n = last if j == ndma - 1 else 128
                pltpu.async_copy(table_hbm.at[idx_v.at[b, pl.ds(j * 128, n)]],
                                 rows_v.at[b, pl.ds(j * 128, n)], sems[b])

        def drain(b, nrows):
            pltpu.make_async_copy(out_hbm.at[0, pl.ds(0, nrows)],
                                  rows_v.at[b, pl.ds(0, nrows)],
                                  sems[b]).wait()

        def writeout(c, b, nrows):
            pltpu.sync_copy(rows_v.at[b, pl.ds(0, nrows)],
                            out_hbm.at[side, pl.ds(base + c * GCH, nrows)])

        fire(0, 0, GCH, GK, 128)

        def pair(i, carry):
            c0 = 2 * i
            fire(c0 + 1, 1, GCH, GK, 128)
            drain(0, GCH)
            writeout(c0, 0, GCH)

            @pl.when(c0 + 2 < GC)
            def _():
                fire(c0 + 2, 0, GCH, GK, 128)

            drain(1, GCH)
            writeout(c0 + 1, 1, GCH)
            return carry

        lax.fori_loop(0, GC // 2, pair, 0)
        fire(GC, 0, GT, GTN + 1, GTR)
        drain(0, GT)
        writeout(GC, 0, GT)

    @functools.partial(
        pl.kernel,
        out_type=jax.ShapeDtypeStruct((2, NP, 8), jnp.float32),
        mesh=mesh,
        scratch_types=[
            pltpu.VMEM((SK, 128), jnp.int32),
            pltpu.VMEM((SK * 128, 8), jnp.float32),
            pltpu.VMEM_SHARED((NP, 8), jnp.float32),
        ],
        compiler_params=pltpu.CompilerParams(use_tc_tiling_on_sc=False),
    )
    def _scatter_k(ep_hbm, cidx_hbm, zero_hbm, out_hbm, idx_v, ep_v, acc):
        cid = lax.axis_index("c")
        sid = lax.axis_index("s")
        w = sid * 2 + cid
        pltpu.sync_copy(zero_hbm.at[pl.ds(sid * NROWS, NROWS)],
                        acc.at[pl.ds(sid * NROWS, NROWS)])
        plsc.subcore_barrier()

        def do_block(c, nidx):
            base = w * SPW + c * (SK * 128)
            pltpu.sync_copy(cidx_hbm.at[w, pl.ds(c * SK, nidx)],
                            idx_v.at[pl.ds(0, nidx)])
            pltpu.sync_copy(ep_hbm.at[pl.ds(base, nidx * 128)],
                            ep_v.at[pl.ds(0, nidx * 128)])
            for j in range(nidx):
                pltpu.sync_copy(ep_v.at[pl.ds(j * 128, 128)],
                                acc.at[idx_v.at[j]], add=True)

        def chunk(c, carry):
            do_block(c, SK)
            return carry

        lax.fori_loop(0, SCH, chunk, 0)
        do_block(SCH, STL)
        plsc.subcore_barrier()
        pltpu.sync_copy(acc.at[pl.ds(sid * NROWS, NROWS)],
                        out_hbm.at[cid, pl.ds(sid * NROWS, NROWS)])

    return _gather_k, _scatter_k


BE = 6400


def _edge_mlp_body(gr_ref, gc_ref, ea_ref, eeW_ref, eeb_ref, W1r_ref, W1c_ref,
                   W1e_ref, b1_ref, W2_ref, b2_ref, W3_ref, b3_ref, out_ref):
    xr = gr_ref[0]
    xc = gc_ref[0]
    xa = ea_ref[...]
    gr = jnp.concatenate([xr[:, 32 * k:32 * k + 32] for k in range(4)],
                         axis=0)
    gc = jnp.concatenate([xc[:, 32 * k:32 * k + 32] for k in range(4)],
                         axis=0)
    ea = jnp.concatenate([xa[:, k:k + 1] for k in range(4)], axis=0)
    eemb = jnp.maximum(ea * eeW_ref[...] + eeb_ref[...], 0.0)
    h = (jnp.dot(gr, W1r_ref[...], preferred_element_type=jnp.float32)
         + jnp.dot(gc, W1c_ref[...], preferred_element_type=jnp.float32)
         + jnp.dot(eemb, W1e_ref[...], preferred_element_type=jnp.float32)
         + b1_ref[...])
    h = jnp.maximum(h, 0.0)
    h = jnp.maximum(
        jnp.dot(h, W2_ref[...], preferred_element_type=jnp.float32)
        + b2_ref[...], 0.0)
    ep = (jnp.dot(h, W3_ref[...], preferred_element_type=jnp.float32)
          + b3_ref[...])
    out_ref[...] = jnp.concatenate(
        [ep[(BE // 16) * j:(BE // 16) * (j + 1)] for j in range(16)],
        axis=1)


def _edge_mlp(gath_pk, ea_pk, eeW, eeb, W1r, W1c, W1e, b1, W2, b2, W3, b3):
    grid = E // BE
    full = lambda a: pl.BlockSpec(a.shape, lambda i: tuple(0 for _ in a.shape))
    return pl.pallas_call(
        _edge_mlp_body,
        grid=(grid,),
        in_specs=[
            pl.BlockSpec((1, BE // 4, 128), lambda i: (0, i, 0)),
            pl.BlockSpec((1, BE // 4, 128), lambda i: (1, i, 0)),
            pl.BlockSpec((BE // 4, 4), lambda i: (i, 0)),
            full(eeW), full(eeb), full(W1r), full(W1c), full(W1e), full(b1),
            full(W2), full(b2), full(W3), full(b3),
        ],
        out_specs=pl.BlockSpec((BE // 16, 128), lambda i: (i, 0)),
        out_shape=jax.ShapeDtypeStruct((EEP // 16, 128), jnp.float32),
    )(gath_pk, gath_pk, ea_pk, eeW, eeb, W1r, W1c, W1e, b1, W2, b2, W3, b3)


def _node_mlp_body(p_ref, W1_ref, b1_ref, W2_ref, b2_ref, Wd_ref, bd_ref,
                   out_ref):
    br = p_ref.shape[1]
    ebpk = p_ref[0] + p_ref[1]
    eb = jnp.concatenate([ebpk[:, 8 * j:8 * j + 8] for j in range(16)],
                         axis=0)
    g = jnp.maximum(
        jnp.dot(eb, W1_ref[...], preferred_element_type=jnp.float32)
        + b1_ref[...], 0.0)
    g = jnp.maximum(
        jnp.dot(g, W2_ref[...], preferred_element_type=jnp.float32)
        + b2_ref[...], 0.0)
    ov = (jnp.dot(g, Wd_ref[...], preferred_element_type=jnp.float32)
          + bd_ref[...])
    out_ref[...] = jnp.concatenate(
        [ov[br * j:br * (j + 1)] for j in range(16)], axis=1)


def _node_mlp(parts, W1, b1, W2, b2, Wd, bd):
    brows = 136
    grid = (NP // 16) // brows
    full = lambda a: pl.BlockSpec(a.shape, lambda i: tuple(0 for _ in a.shape))
    return pl.pallas_call(
        _node_mlp_body,
        grid=(grid,),
        in_specs=[
            pl.BlockSpec((2, brows, 128), lambda i: (0, i, 0)),
            full(W1), full(b1), full(W2), full(b2), full(Wd), full(bd),
        ],
        out_specs=pl.BlockSpec((brows, 16), lambda i: (i, 0)),
        out_shape=jax.ShapeDtypeStruct((N // 16, 16), jnp.float32),
    )(parts, W1, b1, W2, b2, Wd, bd)


def kernel(x, edge_index, batch, node_attr, edge_attr, glob_attr,
           se_W, se_b, ne_W, ne_b, ee_W, ee_b, ge_W, ge_b,
           pe_W1, pe_b1, pe_W2, pe_b2, pe_W3, pe_b3,
           pv_W1, pv_b1, pv_W2, pv_b2, pv_W3, pv_b3,
           dec_W, dec_b):
    colp = (edge_index[1]
            .reshape(E // BE, BE // 4, 4).transpose(0, 2, 1)
            .reshape(E // BE, 16, BE // 16).transpose(0, 2, 1)
            .reshape(NW, SPW))
    cidx = jnp.pad(colp, ((0, 0), (0, SROWS * 128 - SPW)),
                   constant_values=DUMMY)
    cidx = cidx.reshape(NW, SROWS, 128)
    zeros = jnp.zeros((NP, 8), jnp.float32)

    z8 = jnp.zeros((8, pe_W1.shape[1]), jnp.float32)
    W1r = jnp.concatenate([pe_W1[0:24], z8], axis=0)
    W1c = jnp.concatenate([pe_W1[24:48], z8], axis=0)
    W1e = pe_W1[48:56]
    Wd = pv_W3 @ dec_W
    bd = (pv_b3 @ dec_W + dec_b).reshape(1, 1)

    gather_k, scatter_k = _sc_kernels()
    table = _node_encoder(x.reshape(N // 4, 4), node_attr.reshape(N // 4, 4),
                          se_W, se_b, ne_W, ne_b.reshape(1, -1))
    gath = gather_k(table.reshape(N, TW), edge_index)
    ep = _edge_mlp(gath.reshape(2, E // 4, 128),
                   edge_attr.reshape(E // 4, 4),
                   ee_W, ee_b.reshape(1, -1),
                   W1r, W1c, W1e, pe_b1.reshape(1, -1),
                   pe_W2, pe_b2.reshape(1, -1), pe_W3, pe_b3.reshape(1, -1))
    parts = scatter_k(ep.reshape(EEP, 8), cidx, zeros)
    out = _node_mlp(parts.reshape(2, NP // 16, 128), pv_W1,
                    pv_b1.reshape(1, -1), pv_W2, pv_b2.reshape(1, -1), Wd, bd)
    return out.reshape(N, 1)
